# baseline (speedup 1.0000x reference)
import functools

import jax
import jax.numpy as jnp
from jax import lax
from jax.experimental import pallas as pl
from jax.experimental.pallas import tpu as pltpu
from jax.experimental.pallas import tpu_sc as plsc

_B = 4096 * 200
_V = 1000000
_D = 64
_NW = 32
_L = 16

_NCOL = (_V + 127) // 128
_CPW = (_NCOL + _NW - 1) // _NW

_LPW = _B // _NW
_STG = 1024
_NSTG = _LPW // _STG
_CH = 128
_NCH = _STG // _CH

_mesh = plsc.VectorSubcoreMesh(core_axis_name="c", subcore_axis_name="s")
_params = pltpu.CompilerParams(needs_layout_passes=False)


@functools.partial(
    pl.kernel,
    mesh=_mesh,
    out_type=jax.ShapeDtypeStruct((_V // 2, 2 * _D), jnp.float32),
    scratch_types=[
        pltpu.VMEM((_D, 128), jnp.float32),
        pltpu.VMEM((_D, 128), jnp.float32),
    ],
    compiler_params=_params,
)
def _repack_kernel(wt_hbm, w2_hbm, in_v, tr_v):
    wid = lax.axis_index("s") * 2 + lax.axis_index("c")
    lanes = lax.iota(jnp.int32, _L)

    def transpose_cols(ncolgrp):
        def tr_body(e, carry2):
            for ug in range(ncolgrp):
                u = ug * _L + lanes
                val = in_v[e, pl.ds(ug * _L, _L)]
                plsc.store_scatter(tr_v, [u >> 1, (u & 1) * _D + e], val)
            return carry2
        lax.fori_loop(0, _D, tr_body, 0)

    def col_body(i, carry):
        col = wid * _CPW + i

        @pl.when(col < _NCOL - 1)
        def _full():
            pltpu.sync_copy(wt_hbm.at[:, pl.ds(col * 128, 128)], in_v)
            transpose_cols(8)
            pltpu.sync_copy(tr_v, w2_hbm.at[pl.ds(col * _D, _D)])

        return carry

    lax.fori_loop(0, _CPW, col_body, 0)


@functools.partial(
    pl.kernel,
    mesh=_mesh,
    out_type=jax.ShapeDtypeStruct((_B, _D), jnp.float32),
    scratch_types=[
        pltpu.VMEM((_STG,), jnp.int32),
        pltpu.VMEM((_STG,), jnp.int32),
        pltpu.VMEM((_STG,), jnp.int32),
        pltpu.VMEM((_CH, 2 * _D), jnp.float32),
        pltpu.VMEM((_CH, _D), jnp.float32),
        pltpu.SemaphoreType.DMA,
    ],
    compiler_params=_params,
)
def _gather_kernel(x_hbm, w2_hbm, out_hbm, idx_v, row_v, col_v,
                   gbuf, obuf, sem):
    wid = lax.axis_index("s") * 2 + lax.axis_index("c")
    base = wid * _LPW
    lanes = lax.iota(jnp.int32, _L)

    def stage_body(s, carry):
        sbase = base + s * _STG
        pltpu.sync_copy(x_hbm.at[pl.ds(sbase, _STG)], idx_v)

        def split_body(j, carry2):
            v = idx_v[pl.ds(j * _L, _L)]
            row_v[pl.ds(j * _L, _L)] = v >> 1
            col_v[pl.ds(j * _L, _L)] = (v & 1) * _D
            return carry2
        lax.fori_loop(0, _STG // _L, split_body, 0)

        def chunk_body(c, carry2):
            pltpu.async_copy(
                w2_hbm.at[row_v.at[pl.ds(c * _CH, _CH)]], gbuf, sem,
            ).wait()

            def sel_body(r, carry3):
                rows = r * _L + lanes
                cbase = col_v[pl.ds(c * _CH + r * _L, _L)]
                for col in range(_D):
                    val = plsc.load_gather(gbuf, [rows, cbase + col])
                    plsc.store_scatter(
                        obuf, [rows, jnp.full((_L,), col, jnp.int32)], val)
                return carry3
            lax.fori_loop(0, _CH // _L, sel_body, 0)
            pltpu.sync_copy(obuf, out_hbm.at[pl.ds(sbase + c * _CH, _CH)])
            return carry2
        lax.fori_loop(0, _NCH, chunk_body, 0)
        return carry

    lax.fori_loop(0, _NSTG, stage_body, 0)


def kernel(x, weight):
    w2 = _repack_kernel(weight.T)
    tail = weight[(_NCOL - 1) * 128:].reshape(_D // 2, 2 * _D)
    w2 = lax.dynamic_update_slice(w2, tail, ((_NCOL - 1) * _D, 0))
    out = _gather_kernel(x.reshape(_B).astype(jnp.int32), w2)
    return out.reshape(x.shape + (_D,))


# speedup vs baseline: 1.2139x; 1.2139x over previous
import functools

import jax
import jax.numpy as jnp
from jax import lax
from jax.experimental import pallas as pl
from jax.experimental.pallas import tpu as pltpu
from jax.experimental.pallas import tpu_sc as plsc

_B = 4096 * 200
_V = 1000000
_D = 64
_NW = 32
_L = 16

_NCOL = _V // 128
_CPW = (_NCOL + _NW - 1) // _NW
_CPW += _CPW % 2

_LPW = _B // _NW
_STG = 1024
_NSTG = _LPW // _STG
_CH = 128
_NCH = _STG // _CH

_mesh = plsc.VectorSubcoreMesh(core_axis_name="c", subcore_axis_name="s")
_params = pltpu.CompilerParams(needs_layout_passes=False)


@functools.partial(
    pl.kernel,
    mesh=_mesh,
    out_type=jax.ShapeDtypeStruct((_V // 2, 2 * _D), jnp.float32),
    scratch_types=[
        pltpu.VMEM((_D, 128), jnp.float32),
        pltpu.VMEM((_D, 128), jnp.float32),
        pltpu.VMEM((_D, 128), jnp.float32),
        pltpu.VMEM((_D, 128), jnp.float32),
        pltpu.SemaphoreType.DMA,
        pltpu.SemaphoreType.DMA,
        pltpu.SemaphoreType.DMA,
        pltpu.SemaphoreType.DMA,
    ],
    compiler_params=_params,
)
def _repack_kernel(wt_hbm, w2_hbm, in_a, in_b, tr_a, tr_b,
                   rs_a, rs_b, ws_a, ws_b):
    wid = lax.axis_index("s") * 2 + lax.axis_index("c")
    lanes = lax.iota(jnp.int32, _L)
    col0 = wid * _CPW

    def read(col, in_v, rs):
        pltpu.async_copy(wt_hbm.at[:, pl.ds(col * 128, 128)], in_v, rs)

    def transpose(in_v, tr_v):
        def tr_body(e, carry2):
            for ug in range(8):
                u = ug * _L + lanes
                val = in_v[e, pl.ds(ug * _L, _L)]
                plsc.store_scatter(tr_v, [u >> 1, (u & 1) * _D + e], val)
            return carry2
        lax.fori_loop(0, _D, tr_body, 0)

    def process(i, col, in_v, tr_v, rs, ws):
        @pl.when(jnp.logical_and(i > 0, col < _NCOL))
        def _():
            pltpu.make_async_copy(tr_v, w2_hbm.at[pl.ds(0, _D)], ws).wait()
        @pl.when(col < _NCOL)
        def _():
            pltpu.make_async_copy(
                wt_hbm.at[:, pl.ds(0, 128)], in_v, rs).wait()
            transpose(in_v, tr_v)
            pltpu.async_copy(tr_v, w2_hbm.at[pl.ds(col * _D, _D)], ws)

    @pl.when(col0 < _NCOL)
    def _():
        read(col0, in_a, rs_a)

    def body(i, carry):
        col_a = col0 + 2 * i
        col_b = col_a + 1
        @pl.when(col_b < _NCOL)
        def _():
            read(col_b, in_b, rs_b)
        process(i, col_a, in_a, tr_a, rs_a, ws_a)
        @pl.when(jnp.logical_and(i < _CPW // 2 - 1, col_b + 1 < _NCOL))
        def _():
            read(col_b + 1, in_a, rs_a)
        process(i, col_b, in_b, tr_b, rs_b, ws_b)
        return carry

    lax.fori_loop(0, _CPW // 2, body, 0)
    @pl.when(col0 < _NCOL)
    def _():
        pltpu.make_async_copy(tr_a, w2_hbm.at[pl.ds(0, _D)], ws_a).wait()
    @pl.when(col0 + 1 < _NCOL)
    def _():
        pltpu.make_async_copy(tr_b, w2_hbm.at[pl.ds(0, _D)], ws_b).wait()


@functools.partial(
    pl.kernel,
    mesh=_mesh,
    out_type=jax.ShapeDtypeStruct((_B, _D), jnp.float32),
    scratch_types=[
        pltpu.VMEM((_STG,), jnp.int32),
        pltpu.VMEM((_STG,), jnp.int32),
        pltpu.VMEM((_STG,), jnp.int32),
        pltpu.VMEM((_CH, 2 * _D), jnp.float32),
        pltpu.VMEM((_CH, 2 * _D), jnp.float32),
        pltpu.VMEM((_CH, _D), jnp.float32),
        pltpu.VMEM((_CH, _D), jnp.float32),
        pltpu.SemaphoreType.DMA,
        pltpu.SemaphoreType.DMA,
        pltpu.SemaphoreType.DMA,
        pltpu.SemaphoreType.DMA,
    ],
    compiler_params=_params,
)
def _gather_kernel(x_hbm, w2_hbm, out_hbm, idx_v, row_v, col_v,
                   gb_a, gb_b, ob_a, ob_b, gs_a, gs_b, ws_a, ws_b):
    wid = lax.axis_index("s") * 2 + lax.axis_index("c")
    base = wid * _LPW
    lanes = lax.iota(jnp.int32, _L)

    def fire(c, gbuf, gs):
        pltpu.async_copy(
            w2_hbm.at[row_v.at[pl.ds(c * _CH, _CH)]], gbuf, gs)

    def select(c, gbuf, obuf):
        def sel_body(r, carry3):
            rows = r * _L + lanes
            cbase = col_v[pl.ds(c * _CH + r * _L, _L)]
            for col in range(_D):
                val = plsc.load_gather(gbuf, [rows, cbase + col])
                plsc.store_scatter(
                    obuf, [rows, jnp.full((_L,), col, jnp.int32)], val)
            return carry3
        lax.fori_loop(0, _CH // _L, sel_body, 0)

    def stage_body(s, carry):
        sbase = base + s * _STG
        pltpu.sync_copy(x_hbm.at[pl.ds(sbase, _STG)], idx_v)

        def split_body(j, carry2):
            v = idx_v[pl.ds(j * _L, _L)]
            row_v[pl.ds(j * _L, _L)] = v >> 1
            col_v[pl.ds(j * _L, _L)] = (v & 1) * _D
            return carry2
        lax.fori_loop(0, _STG // _L, split_body, 0)

        fire(0, gb_a, gs_a)

        def pair_body(i, carry2):
            c_a = 2 * i
            c_b = 2 * i + 1
            pltpu.make_async_copy(
                w2_hbm.at[row_v.at[pl.ds(0, _CH)]], gb_a, gs_a).wait()
            fire(c_b, gb_b, gs_b)
            @pl.when(i > 0)
            def _():
                pltpu.make_async_copy(
                    ob_a, out_hbm.at[pl.ds(0, _CH)], ws_a).wait()
            select(c_a, gb_a, ob_a)
            pltpu.async_copy(
                ob_a, out_hbm.at[pl.ds(sbase + c_a * _CH, _CH)], ws_a)
            pltpu.make_async_copy(
                w2_hbm.at[row_v.at[pl.ds(0, _CH)]], gb_b, gs_b).wait()
            @pl.when(c_b + 1 < _NCH)
            def _():
                fire(c_b + 1, gb_a, gs_a)
            @pl.when(i > 0)
            def _():
                pltpu.make_async_copy(
                    ob_b, out_hbm.at[pl.ds(0, _CH)], ws_b).wait()
            select(c_b, gb_b, ob_b)
            pltpu.async_copy(
                ob_b, out_hbm.at[pl.ds(sbase + c_b * _CH, _CH)], ws_b)
            return carry2

        lax.fori_loop(0, _NCH // 2, pair_body, 0)
        pltpu.make_async_copy(ob_a, out_hbm.at[pl.ds(0, _CH)], ws_a).wait()
        pltpu.make_async_copy(ob_b, out_hbm.at[pl.ds(0, _CH)], ws_b).wait()
        return carry

    lax.fori_loop(0, _NSTG, stage_body, 0)


def kernel(x, weight):
    w2 = _repack_kernel(weight.T)
    tail = weight[_NCOL * 128:].reshape(_D // 2, 2 * _D)
    w2 = lax.dynamic_update_slice(w2, tail, (_NCOL * _D, 0))
    out = _gather_kernel(x.reshape(_B).astype(jnp.int32), w2)
    return out.reshape(x.shape + (_D,))


# speedup vs baseline: 3.2613x; 2.6867x over previous
import functools

import jax
import jax.numpy as jnp
from jax import lax
from jax.experimental import pallas as pl
from jax.experimental.pallas import tpu as pltpu
from jax.experimental.pallas import tpu_sc as plsc

_R = 4096
_C = 200
_D = 64
_NW = 32
_RPW = _R // _NW
_G = 4
_NGRP = _RPW // _G
_NIT = _NGRP // 2
_SPLITS = ((0, 128), (128, 72))

_mesh = plsc.VectorSubcoreMesh(core_axis_name="c", subcore_axis_name="s")


@functools.partial(
    pl.kernel,
    mesh=_mesh,
    out_type=jax.ShapeDtypeStruct((_R, _C, _D), jnp.float32),
    scratch_types=[
        pltpu.VMEM((2 * _G, _C), jnp.int32),
        pltpu.VMEM((_G, _C, _D), jnp.float32),
        pltpu.VMEM((_G, _C, _D), jnp.float32),
        pltpu.SemaphoreType.DMA,
        pltpu.SemaphoreType.DMA,
        pltpu.SemaphoreType.DMA,
    ],
    compiler_params=pltpu.CompilerParams(use_tc_tiling_on_sc=False),
)
def _gather_kernel(x_hbm, w_hbm, out_hbm, idx_v, rows0_v, rows1_v,
                   gsem, wsem0, wsem1):
    wid = lax.axis_index("s") * 2 + lax.axis_index("c")
    row0 = wid * _RPW

    def fire(iofs, rows_v):
        copies = []
        for g in range(_G):
            for (lo, n) in _SPLITS:
                copies.append(pltpu.async_copy(
                    w_hbm.at[idx_v.at[iofs + g, pl.ds(lo, n)]],
                    rows_v.at[g, pl.ds(lo, n)],
                    gsem,
                ))
        return copies

    def body(i, carry):
        r_a = row0 + 2 * i * _G
        pltpu.sync_copy(x_hbm.at[pl.ds(r_a, 2 * _G)], idx_v)
        @pl.when(i > 0)
        def _():
            pltpu.make_async_copy(
                rows0_v, out_hbm.at[pl.ds(0, _G)], wsem0).wait()
        ca = fire(0, rows0_v)
        @pl.when(i > 0)
        def _():
            pltpu.make_async_copy(
                rows1_v, out_hbm.at[pl.ds(0, _G)], wsem1).wait()
        for c in ca:
            c.wait()
        pltpu.async_copy(rows0_v, out_hbm.at[pl.ds(r_a, _G)], wsem0)
        cb = fire(_G, rows1_v)
        for c in cb:
            c.wait()
        pltpu.async_copy(rows1_v, out_hbm.at[pl.ds(r_a + _G, _G)], wsem1)
        return carry

    lax.fori_loop(0, _NIT, body, 0)
    pltpu.make_async_copy(rows0_v, out_hbm.at[pl.ds(0, _G)], wsem0).wait()
    pltpu.make_async_copy(rows1_v, out_hbm.at[pl.ds(0, _G)], wsem1).wait()


def kernel(x, weight):
    return _gather_kernel(x.astype(jnp.int32), weight)
